# baseline (speedup 1.0000x reference)
import functools

import jax
import jax.numpy as jnp
from jax import lax
from jax.experimental import pallas as pl
from jax.experimental.pallas import tpu as pltpu
from jax.experimental.pallas import tpu_sc as plsc

_VOCAB = 100000
_EMB = 128
_HID = 128
_CTX = 200

_VT = 33792
_NT = -(-_VOCAB // _VT)
_TAIL = _VOCAB - (_NT - 1) * _VT
_FULL = 16
_NFULL = _CTX // _FULL
_REM = _CTX - _NFULL * _FULL
_NW = _NFULL + 1


def _sc_sum_rows(rows_v, acc_v, n):
    for ch in range(_EMB // 16):
        v = rows_v.at[0][pl.ds(ch * 16, 16)]
        for r in range(1, n):
            v = v + rows_v.at[r][pl.ds(ch * 16, 16)]
        acc_v[0, pl.ds(ch * 16, 16)] = v


def _sc_gather_body(idx_hbm, emb_hbm, out_hbm, idx_v, rows_v, acc_v, sem):
    wid = lax.axis_index("s")

    @pl.when(wid < _NFULL)
    def _():
        pltpu.sync_copy(idx_hbm.at[pl.ds(wid * _FULL, _FULL)], idx_v)
        pltpu.async_copy(emb_hbm.at[idx_v], rows_v, sem).wait()
        _sc_sum_rows(rows_v, acc_v, _FULL)

    @pl.when(wid == _NFULL)
    def _():
        pltpu.sync_copy(idx_hbm.at[pl.ds(_NFULL * _FULL, _REM)],
                        idx_v.at[pl.ds(0, _REM)])
        pltpu.async_copy(emb_hbm.at[idx_v.at[pl.ds(0, _REM)]],
                         rows_v.at[pl.ds(0, _REM)], sem).wait()
        _sc_sum_rows(rows_v, acc_v, _REM)

    @pl.when(wid <= _NFULL)
    def _():
        pltpu.sync_copy(acc_v, out_hbm.at[pl.ds(wid, 1)])


_sc_gather = functools.partial(
    pl.kernel,
    out_type=jax.ShapeDtypeStruct((_NW, _EMB), jnp.float32),
    mesh=plsc.VectorSubcoreMesh(
        core_axis_name="c", subcore_axis_name="s", num_cores=1),
    scratch_types=[
        pltpu.VMEM((_FULL,), jnp.int32),
        pltpu.VMEM((_FULL, _EMB), jnp.float32),
        pltpu.VMEM((1, _EMB), jnp.float32),
        pltpu.SemaphoreType.DMA,
    ],
)(_sc_gather_body)


def _main_body(parts_ref, w1_ref, b1_ref, w2_ref, b2_ref,
               out_ref, h_ref, m_ref, s_ref):
    i = pl.program_id(0)

    @pl.when(i == 0)
    def _():
        embeds = jnp.sum(parts_ref[...], axis=0, keepdims=True)
        pre = lax.dot_general(
            embeds, w1_ref[...], (((1,), (1,)), ((), ())),
            preferred_element_type=jnp.float32) + b1_ref[...].reshape(1, _HID)
        h_ref[...] = jnp.maximum(pre, 0.0)
        m_ref[0] = -jnp.inf
        s_ref[0] = 0.0

    logits = lax.dot_general(
        h_ref[...], w2_ref[...], (((1,), (1,)), ((), ())),
        preferred_element_type=jnp.float32) + b2_ref[...].reshape(1, _VT)

    @pl.when(i < _NT - 1)
    def _():
        out_ref[:, pl.ds(pl.multiple_of(i * _VT, _VT), _VT)] = logits

    @pl.when(i == _NT - 1)
    def _():
        out_ref[:, pl.ds(_VOCAB - _TAIL, _TAIL)] = logits[:, :_TAIL]

    lane = lax.broadcasted_iota(jnp.int32, (1, _VT), 1)
    valid = (i * _VT + lane) < _VOCAB
    logits_m = jnp.where(valid, logits, -jnp.inf)

    tile_max = jnp.max(logits_m)
    m_old = m_ref[0]
    m_new = jnp.maximum(m_old, tile_max)
    s_ref[0] = s_ref[0] * jnp.exp(m_old - m_new) + jnp.sum(
        jnp.where(valid, jnp.exp(logits_m - m_new), 0.0))
    m_ref[0] = m_new

    @pl.when(i == _NT - 1)
    def _():
        lse = m_new + jnp.log(s_ref[0])
        out_ref[...] = out_ref[...] - lse


def kernel(inputs, emb, W1, b1, W2, b2):
    idx = inputs.astype(jnp.int32)
    parts = _sc_gather(idx, emb)

    log_probs = pl.pallas_call(
        _main_body,
        grid=(_NT,),
        in_specs=[
            pl.BlockSpec((_NW, _EMB), lambda i: (0, 0)),
            pl.BlockSpec((_HID, _EMB), lambda i: (0, 0)),
            pl.BlockSpec((_HID,), lambda i: (0,)),
            pl.BlockSpec((_VT, _HID), lambda i: (i, 0)),
            pl.BlockSpec((_VT,), lambda i: (i,)),
        ],
        out_specs=pl.BlockSpec((1, _VOCAB), lambda i: (0, 0)),
        out_shape=jax.ShapeDtypeStruct((1, _VOCAB), jnp.float32),
        scratch_shapes=[
            pltpu.VMEM((1, _HID), jnp.float32),
            pltpu.SMEM((1,), jnp.float32),
            pltpu.SMEM((1,), jnp.float32),
        ],
    )(parts, W1, b1, W2, b2)

    return log_probs


# speedup vs baseline: 1.0220x; 1.0220x over previous
import functools

import jax
import jax.numpy as jnp
from jax import lax
from jax.experimental import pallas as pl
from jax.experimental.pallas import tpu as pltpu
from jax.experimental.pallas import tpu_sc as plsc

_VOCAB = 100000
_EMB = 128
_HID = 128
_CTX = 200

_VT = 20480
_NT = -(-_VOCAB // _VT)
_TAIL = _VOCAB - (_NT - 1) * _VT
_FULL = 16
_NFULL = _CTX // _FULL
_REM = _CTX - _NFULL * _FULL
_NW = _NFULL + 1


def _sc_sum_rows(rows_v, acc_v, n):
    for ch in range(_EMB // 16):
        v = rows_v.at[0][pl.ds(ch * 16, 16)]
        for r in range(1, n):
            v = v + rows_v.at[r][pl.ds(ch * 16, 16)]
        acc_v[0, pl.ds(ch * 16, 16)] = v


def _sc_gather_body(idx_hbm, emb_hbm, out_hbm, idx_v, rows_v, acc_v, sem):
    wid = lax.axis_index("s")

    @pl.when(wid < _NFULL)
    def _():
        pltpu.sync_copy(idx_hbm.at[pl.ds(wid * _FULL, _FULL)], idx_v)
        pltpu.async_copy(emb_hbm.at[idx_v], rows_v, sem).wait()
        _sc_sum_rows(rows_v, acc_v, _FULL)

    @pl.when(wid == _NFULL)
    def _():
        pltpu.sync_copy(idx_hbm.at[pl.ds(_NFULL * _FULL, _REM)],
                        idx_v.at[pl.ds(0, _REM)])
        pltpu.async_copy(emb_hbm.at[idx_v.at[pl.ds(0, _REM)]],
                         rows_v.at[pl.ds(0, _REM)], sem).wait()
        _sc_sum_rows(rows_v, acc_v, _REM)

    @pl.when(wid <= _NFULL)
    def _():
        pltpu.sync_copy(acc_v, out_hbm.at[pl.ds(wid, 1)])


_sc_gather = functools.partial(
    pl.kernel,
    out_type=jax.ShapeDtypeStruct((_NW, _EMB), jnp.float32),
    mesh=plsc.VectorSubcoreMesh(
        core_axis_name="c", subcore_axis_name="s", num_cores=1),
    scratch_types=[
        pltpu.VMEM((_FULL,), jnp.int32),
        pltpu.VMEM((_FULL, _EMB), jnp.float32),
        pltpu.VMEM((1, _EMB), jnp.float32),
        pltpu.SemaphoreType.DMA,
    ],
)(_sc_gather_body)


def _main_body(parts_ref, w1_ref, b1_ref, w2_ref, b2_ref,
               out_ref, h_ref, m_ref, s_ref):
    i = pl.program_id(0)

    @pl.when(i == 0)
    def _():
        embeds = jnp.sum(parts_ref[...], axis=0, keepdims=True)
        pre = lax.dot_general(
            embeds, w1_ref[...], (((1,), (1,)), ((), ())),
            preferred_element_type=jnp.float32) + b1_ref[...].reshape(1, _HID)
        h_ref[...] = jnp.maximum(pre, 0.0)
        m_ref[0] = -jnp.inf
        s_ref[0] = 0.0

    logits = lax.dot_general(
        h_ref[...], w2_ref[...], (((1,), (1,)), ((), ())),
        preferred_element_type=jnp.float32) + b2_ref[...].reshape(1, _VT)

    @pl.when(i < _NT - 1)
    def _():
        out_ref[:, pl.ds(pl.multiple_of(i * _VT, _VT), _VT)] = logits

    @pl.when(i == _NT - 1)
    def _():
        out_ref[:, pl.ds(_VOCAB - _TAIL, _TAIL)] = logits[:, :_TAIL]

    lane = lax.broadcasted_iota(jnp.int32, (1, _VT), 1)
    valid = (i * _VT + lane) < _VOCAB
    logits_m = jnp.where(valid, logits, -jnp.inf)

    tile_max = jnp.max(logits_m)
    m_old = m_ref[0]
    m_new = jnp.maximum(m_old, tile_max)
    s_ref[0] = s_ref[0] * jnp.exp(m_old - m_new) + jnp.sum(
        jnp.where(valid, jnp.exp(logits_m - m_new), 0.0))
    m_ref[0] = m_new

    @pl.when(i == _NT - 1)
    def _():
        lse = m_new + jnp.log(s_ref[0])
        out_ref[...] = out_ref[...] - lse


def kernel(inputs, emb, W1, b1, W2, b2):
    idx = inputs.astype(jnp.int32)
    parts = _sc_gather(idx, emb)

    log_probs = pl.pallas_call(
        _main_body,
        grid=(_NT,),
        in_specs=[
            pl.BlockSpec((_NW, _EMB), lambda i: (0, 0)),
            pl.BlockSpec((_HID, _EMB), lambda i: (0, 0)),
            pl.BlockSpec((_HID,), lambda i: (0,)),
            pl.BlockSpec((_VT, _HID), lambda i: (i, 0)),
            pl.BlockSpec((_VT,), lambda i: (i,)),
        ],
        out_specs=pl.BlockSpec((1, _VOCAB), lambda i: (0, 0)),
        out_shape=jax.ShapeDtypeStruct((1, _VOCAB), jnp.float32),
        scratch_shapes=[
            pltpu.VMEM((1, _HID), jnp.float32),
            pltpu.SMEM((1,), jnp.float32),
            pltpu.SMEM((1,), jnp.float32),
        ],
    )(parts, W1, b1, W2, b2)

    return log_probs


# speedup vs baseline: 1.0242x; 1.0022x over previous
import functools

import jax
import jax.numpy as jnp
from jax import lax
from jax.experimental import pallas as pl
from jax.experimental.pallas import tpu as pltpu
from jax.experimental.pallas import tpu_sc as plsc

_VOCAB = 100000
_EMB = 128
_HID = 128
_CTX = 200

_VT = 25600
_NT = -(-_VOCAB // _VT)
_TAIL = _VOCAB - (_NT - 1) * _VT
_FULL = 16
_NFULL = _CTX // _FULL
_REM = _CTX - _NFULL * _FULL
_NW = _NFULL + 1


def _sc_sum_rows(rows_v, acc_v, n):
    for ch in range(_EMB // 16):
        v = rows_v.at[0][pl.ds(ch * 16, 16)]
        for r in range(1, n):
            v = v + rows_v.at[r][pl.ds(ch * 16, 16)]
        acc_v[0, pl.ds(ch * 16, 16)] = v


def _sc_gather_body(idx_hbm, emb_hbm, out_hbm, idx_v, rows_v, acc_v, sem):
    wid = lax.axis_index("s")

    @pl.when(wid < _NFULL)
    def _():
        pltpu.sync_copy(idx_hbm.at[pl.ds(wid * _FULL, _FULL)], idx_v)
        pltpu.async_copy(emb_hbm.at[idx_v], rows_v, sem).wait()
        _sc_sum_rows(rows_v, acc_v, _FULL)

    @pl.when(wid == _NFULL)
    def _():
        pltpu.sync_copy(idx_hbm.at[pl.ds(_NFULL * _FULL, _REM)],
                        idx_v.at[pl.ds(0, _REM)])
        pltpu.async_copy(emb_hbm.at[idx_v.at[pl.ds(0, _REM)]],
                         rows_v.at[pl.ds(0, _REM)], sem).wait()
        _sc_sum_rows(rows_v, acc_v, _REM)

    @pl.when(wid <= _NFULL)
    def _():
        pltpu.sync_copy(acc_v, out_hbm.at[pl.ds(wid, 1)])


_sc_gather = functools.partial(
    pl.kernel,
    out_type=jax.ShapeDtypeStruct((_NW, _EMB), jnp.float32),
    mesh=plsc.VectorSubcoreMesh(
        core_axis_name="c", subcore_axis_name="s", num_cores=1),
    scratch_types=[
        pltpu.VMEM((_FULL,), jnp.int32),
        pltpu.VMEM((_FULL, _EMB), jnp.float32),
        pltpu.VMEM((1, _EMB), jnp.float32),
        pltpu.SemaphoreType.DMA,
    ],
)(_sc_gather_body)


def _main_body(parts_ref, w1_ref, b1_ref, w2_ref, b2_ref,
               out_ref, h_ref, m_ref, s_ref):
    i = pl.program_id(0)

    @pl.when(i == 0)
    def _():
        embeds = jnp.sum(parts_ref[...], axis=0, keepdims=True)
        pre = lax.dot_general(
            embeds, w1_ref[...], (((1,), (1,)), ((), ())),
            preferred_element_type=jnp.float32) + b1_ref[...].reshape(1, _HID)
        h_ref[...] = jnp.maximum(pre, 0.0)
        m_ref[0] = -jnp.inf
        s_ref[0] = 0.0

    logits = lax.dot_general(
        h_ref[...], w2_ref[...], (((1,), (1,)), ((), ())),
        preferred_element_type=jnp.float32) + b2_ref[...].reshape(1, _VT)

    @pl.when(i < _NT - 1)
    def _():
        out_ref[:, pl.ds(pl.multiple_of(i * _VT, _VT), _VT)] = logits

    @pl.when(i == _NT - 1)
    def _():
        out_ref[:, pl.ds(_VOCAB - _TAIL, _TAIL)] = logits[:, :_TAIL]

    lane = lax.broadcasted_iota(jnp.int32, (1, _VT), 1)
    valid = (i * _VT + lane) < _VOCAB
    logits_m = jnp.where(valid, logits, -jnp.inf)

    tile_max = jnp.max(logits_m)
    m_old = m_ref[0]
    m_new = jnp.maximum(m_old, tile_max)
    s_ref[0] = s_ref[0] * jnp.exp(m_old - m_new) + jnp.sum(
        jnp.where(valid, jnp.exp(logits_m - m_new), 0.0))
    m_ref[0] = m_new

    @pl.when(i == _NT - 1)
    def _():
        lse = m_new + jnp.log(s_ref[0])
        out_ref[...] = out_ref[...] - lse


def kernel(inputs, emb, W1, b1, W2, b2):
    idx = inputs.astype(jnp.int32)
    parts = _sc_gather(idx, emb)

    log_probs = pl.pallas_call(
        _main_body,
        grid=(_NT,),
        in_specs=[
            pl.BlockSpec((_NW, _EMB), lambda i: (0, 0)),
            pl.BlockSpec((_HID, _EMB), lambda i: (0, 0)),
            pl.BlockSpec((_HID,), lambda i: (0,)),
            pl.BlockSpec((_VT, _HID), lambda i: (i, 0)),
            pl.BlockSpec((_VT,), lambda i: (i,)),
        ],
        out_specs=pl.BlockSpec((1, _VOCAB), lambda i: (0, 0)),
        out_shape=jax.ShapeDtypeStruct((1, _VOCAB), jnp.float32),
        scratch_shapes=[
            pltpu.VMEM((1, _HID), jnp.float32),
            pltpu.SMEM((1,), jnp.float32),
            pltpu.SMEM((1,), jnp.float32),
        ],
    )(parts, W1, b1, W2, b2)

    return log_probs
